# baseline (speedup 1.0000x reference)
import functools

import jax
import jax.numpy as jnp
from jax.experimental import pallas as pl
from jax.experimental.pallas import tpu as pltpu

N = 10000
E = 320000
D = 128
H = 64
C = 2

LSTM_BLK = 8


def _prep_body(x_ref, w_ref, asrc_ref, adst_ref, h_ref, al_src_ref, al_dst_ref):
    h = jnp.dot(x_ref[...], w_ref[...], preferred_element_type=jnp.float32)
    h_ref[...] = h
    al_src_ref[...] = h @ asrc_ref[...]
    al_dst_ref[...] = h @ adst_ref[...]


def _prep(x, W, a_src, a_dst):
    n, _ = x.shape
    return pl.pallas_call(
        _prep_body,
        out_shape=(
            jax.ShapeDtypeStruct((n, H), jnp.float32),
            jax.ShapeDtypeStruct((n, 1), jnp.float32),
            jax.ShapeDtypeStruct((n, 1), jnp.float32),
        ),
    )(x, W, a_src[:, None], a_dst[:, None])


def _gat_edges(h, alpha_src, alpha_dst, src, dst, b):
    n = h.shape[0]
    e = jax.nn.leaky_relu(alpha_src[src] + alpha_dst[dst], negative_slope=0.2)
    emax = jax.ops.segment_max(e, dst, num_segments=n)
    emax = jnp.where(jnp.isfinite(emax), emax, 0.0)
    w = jnp.exp(e - emax[dst])
    denom = jax.ops.segment_sum(w, dst, num_segments=n)
    alpha = w / (denom[dst] + 1e-16)
    out = jax.ops.segment_sum(alpha[:, None] * h[src], dst, num_segments=n)
    return jax.nn.relu(out + b)


def _lstm_head_body(h_ref, wih_ref, whh_ref, b_ref, wa_ref, ba_ref, wf_ref,
                    bf_ref, out_ref, g_ref, hs_ref):
    n = h_ref.shape[0]
    g_ref[...] = (
        jnp.dot(h_ref[...], wih_ref[...], preferred_element_type=jnp.float32)
        + b_ref[...]
    )
    whh = whh_ref[...]

    def step(carry, g):
        hprev, cprev = carry
        hcol = hprev.T
        bc = jax.lax.broadcast_in_dim(hcol, (H, 4 * H), (0, 1))
        gates = g + jnp.sum(whh * bc, axis=0, keepdims=True)
        i = jax.nn.sigmoid(gates[:, 0:H])
        f = jax.nn.sigmoid(gates[:, H:2 * H])
        gg = jnp.tanh(gates[:, 2 * H:3 * H])
        o = jax.nn.sigmoid(gates[:, 3 * H:4 * H])
        cnew = f * cprev + i * gg
        hnew = o * jnp.tanh(cnew)
        return (hnew, cnew), hnew

    def blk(k, carry):
        gblk = g_ref[pl.ds(k * LSTM_BLK, LSTM_BLK), :]
        rows = []
        for j in range(LSTM_BLK):
            carry, hnew = step(carry, gblk[j:j + 1, :])
            rows.append(hnew)
        hs_ref[pl.ds(k * LSTM_BLK, LSTM_BLK), :] = jnp.concatenate(rows, axis=0)
        return carry

    zero = jnp.zeros((1, H), jnp.float32)
    jax.lax.fori_loop(0, n // LSTM_BLK, blk, (zero, zero), unroll=False)

    hs = hs_ref[...]
    scores = jnp.dot(hs, wa_ref[...], preferred_element_type=jnp.float32) + ba_ref[0, 0]
    m = jnp.max(scores)
    wexp = jnp.exp(scores - m)
    denom = jnp.sum(wexp)
    pooled = jnp.dot(wexp.T, hs, preferred_element_type=jnp.float32) / denom
    logits = jnp.dot(pooled, wf_ref[...], preferred_element_type=jnp.float32) + bf_ref[...]
    lmax = jnp.max(logits, axis=1, keepdims=True)
    lexp = jnp.exp(logits - lmax)
    out_ref[...] = lexp / jnp.sum(lexp, axis=1, keepdims=True)


def _lstm_head(h, WihT, WhhT, bsum, Wa, ba, Wf, bf):
    n = h.shape[0]
    return pl.pallas_call(
        _lstm_head_body,
        out_shape=jax.ShapeDtypeStruct((1, C), jnp.float32),
        scratch_shapes=[
            pltpu.VMEM((n, 4 * H), jnp.float32),
            pltpu.VMEM((n, H), jnp.float32),
        ],
    )(h, WihT, WhhT, bsum, Wa, ba, Wf, bf)


def kernel(x, edge_index, W1, a_src1, a_dst1, b1, W2, a_src2, a_dst2, b2,
           Wih, Whh, bih, bhh, Wa, ba, Wf, bf):
    src = edge_index[0]
    dst = edge_index[1]

    h1, asrc1, adst1 = _prep(x, W1, a_src1, a_dst1)
    h1o = _gat_edges(h1, asrc1[:, 0], adst1[:, 0], src, dst, b1)

    h2, asrc2, adst2 = _prep(h1o, W2, a_src2, a_dst2)
    h2o = _gat_edges(h2, asrc2[:, 0], adst2[:, 0], src, dst, b2)

    bsum = (bih + bhh)[None, :]
    out = _lstm_head(h2o, Wih.T, Whh.T, bsum, Wa, ba[:, None], Wf, bf[None, :])
    return out


# speedup vs baseline: 6.6667x; 6.6667x over previous
import dataclasses
import functools

import jax
import jax.numpy as jnp
from jax import lax
from jax.experimental import pallas as pl
from jax.experimental.pallas import tpu as pltpu
from jax.experimental.pallas import tpu_sc as plsc

N = 10000
E = 320000
D = 128
H = 64
C = 2

LSTM_BLK = 8

NC = 2
NS = 16
NW = NC * NS
EPT = E // NW
CH = 400
NPD = 10240


def _combine(acc_ref, den_ref, b_ref, n):
    acc = acc_ref[0, :n] + acc_ref[1, :n]
    den = den_ref[0, :n] + den_ref[1, :n]
    hin = acc / (den[:, None] + 1e-16) + b_ref[...]
    return jnp.maximum(hin, 0.0)


def _prep1_body(x_ref, w_ref, asrc_ref, adst_ref, h_ref, al_src_ref,
                al_dst_ref, m_ref):
    h = jnp.dot(x_ref[...], w_ref[...], preferred_element_type=jnp.float32)
    h_ref[...] = h
    als = h @ asrc_ref[...]
    ald = h @ adst_ref[...]
    al_src_ref[...] = als
    al_dst_ref[...] = ald
    s = jnp.max(als) + jnp.max(ald)
    m_ref[...] = jnp.full((1, 16), jnp.maximum(s, 0.2 * s), jnp.float32)


def _prep1(x, W, a_src, a_dst):
    n, _ = x.shape
    return pl.pallas_call(
        _prep1_body,
        out_shape=(
            jax.ShapeDtypeStruct((n, H), jnp.float32),
            jax.ShapeDtypeStruct((n, 1), jnp.float32),
            jax.ShapeDtypeStruct((n, 1), jnp.float32),
            jax.ShapeDtypeStruct((1, 16), jnp.float32),
        ),
    )(x, W, a_src[:, None], a_dst[:, None])


def _prep2_body(acc_ref, den_ref, b_ref, w_ref, asrc_ref, adst_ref, h_ref,
                al_src_ref, al_dst_ref, m_ref):
    hin = _combine(acc_ref, den_ref, b_ref, NPD)
    h = jnp.dot(hin, w_ref[...], preferred_element_type=jnp.float32)
    h_ref[...] = h
    als = h @ asrc_ref[...]
    ald = h @ adst_ref[...]
    al_src_ref[...] = als
    al_dst_ref[...] = ald
    s = jnp.max(als) + jnp.max(ald)
    m_ref[...] = jnp.full((1, 16), jnp.maximum(s, 0.2 * s), jnp.float32)


def _prep2(acc, den, b, W, a_src, a_dst):
    return pl.pallas_call(
        _prep2_body,
        out_shape=(
            jax.ShapeDtypeStruct((NPD, H), jnp.float32),
            jax.ShapeDtypeStruct((NPD, 1), jnp.float32),
            jax.ShapeDtypeStruct((NPD, 1), jnp.float32),
            jax.ShapeDtypeStruct((1, 16), jnp.float32),
        ),
    )(acc, den, b[None, :], W, a_src[:, None], a_dst[:, None])


def _sc_compiler_params():
    cp = pltpu.CompilerParams()
    if "needs_layout_passes" in pltpu.CompilerParams.__dataclass_fields__:
        cp = dataclasses.replace(cp, needs_layout_passes=False,
                                 use_tc_tiling_on_sc=False)
    return cp


def _sc_gat_edges(src, dst, asrc, adst, h, m16, z64, z1):
    mesh = plsc.VectorSubcoreMesh(core_axis_name="c", subcore_axis_name="s")

    @functools.partial(
        pl.kernel,
        compiler_params=_sc_compiler_params(),
        out_type=(
            jax.ShapeDtypeStruct((NC, NPD, H), jnp.float32),
            jax.ShapeDtypeStruct((NC, NPD), jnp.float32),
        ),
        mesh=mesh,
        scratch_types=[
            pltpu.VMEM((NPD,), jnp.float32),
            pltpu.VMEM((NPD,), jnp.float32),
            pltpu.VMEM((16,), jnp.float32),
            pltpu.VMEM((CH,), jnp.int32),
            pltpu.VMEM((CH,), jnp.int32),
            pltpu.VMEM((CH,), jnp.float32),
            pltpu.VMEM((CH, H), jnp.float32),
            pltpu.VMEM_SHARED((NPD, H), jnp.float32),
            pltpu.VMEM_SHARED((NPD,), jnp.float32),
            pltpu.SemaphoreType.DMA,
        ],
    )
    def k(src_hbm, dst_hbm, asrc_hbm, adst_hbm, h_hbm, m_hbm, z64_hbm, z1_hbm,
          acc_out, den_out, asrc_v, adst_v, m_v, srcv, dstv, wbuf, rows_v,
          acc_sh, den_sh, sem):
        cid = lax.axis_index("c")
        sid = lax.axis_index("s")
        g = cid * NS + sid
        rpt = NPD // NS
        dpt = NPD // NS

        pltpu.sync_copy(asrc_hbm, asrc_v)
        pltpu.sync_copy(adst_hbm, adst_v)
        pltpu.sync_copy(m_hbm, m_v)
        pltpu.sync_copy(z64_hbm.at[pl.ds(sid * rpt, rpt)],
                        acc_sh.at[pl.ds(sid * rpt, rpt)])
        pltpu.sync_copy(z1_hbm.at[pl.ds(sid * dpt, dpt)],
                        den_sh.at[pl.ds(sid * dpt, dpt)])
        plsc.subcore_barrier()

        mv = m_v[...]

        @pl.loop(0, EPT // CH)
        def _chunk(c):
            base = g * EPT + c * CH
            pltpu.sync_copy(src_hbm.at[pl.ds(base, CH)], srcv)
            pltpu.sync_copy(dst_hbm.at[pl.ds(base, CH)], dstv)
            gat = pltpu.async_copy(h_hbm.at[srcv], rows_v, sem)

            @pl.loop(0, CH, step=16)
            def _w(j):
                sv = srcv[pl.ds(j, 16)]
                dv = dstv[pl.ds(j, 16)]
                s = (plsc.load_gather(asrc_v, [sv])
                     + plsc.load_gather(adst_v, [dv]))
                e = jnp.maximum(s, 0.2 * s)
                wbuf[pl.ds(j, 16)] = jnp.exp(e - mv)

            gat.wait()

            @pl.loop(0, CH)
            def _scale(r):
                wv = plsc.load_gather(wbuf, [jnp.full((16,), r, jnp.int32)])
                for kk in range(H // 16):
                    rows_v[r, pl.ds(kk * 16, 16)] = (
                        rows_v[r, pl.ds(kk * 16, 16)] * wv)

            pltpu.sync_copy(rows_v, acc_sh.at[dstv], add=True)
            pltpu.sync_copy(wbuf, den_sh.at[dstv], add=True)

        plsc.subcore_barrier()
        pltpu.sync_copy(acc_sh.at[pl.ds(sid * rpt, rpt)],
                        acc_out.at[cid, pl.ds(sid * rpt, rpt)])
        pltpu.sync_copy(den_sh.at[pl.ds(sid * dpt, dpt)],
                        den_out.at[cid, pl.ds(sid * dpt, dpt)])

    return k(src, dst, asrc, adst, h, m16, z64, z1)


def _lstm_head_body(acc_ref, den_ref, b2_ref, wih_ref, whh_ref, b_ref, wa_ref,
                    ba_ref, wf_ref, bf_ref, out_ref, g_ref, hs_ref):
    hin = _combine(acc_ref, den_ref, b2_ref, N)
    g_ref[...] = (
        jnp.dot(hin, wih_ref[...], preferred_element_type=jnp.float32)
        + b_ref[...]
    )
    whh = whh_ref[...]

    def step(carry, g):
        hprev, cprev = carry
        gates = g + jnp.dot(hprev, whh, preferred_element_type=jnp.float32)
        i = jax.nn.sigmoid(gates[:, 0:H])
        f = jax.nn.sigmoid(gates[:, H:2 * H])
        gg = jnp.tanh(gates[:, 2 * H:3 * H])
        o = jax.nn.sigmoid(gates[:, 3 * H:4 * H])
        cnew = f * cprev + i * gg
        hnew = o * jnp.tanh(cnew)
        return (hnew, cnew), hnew

    def blk(k, carry):
        gblk = g_ref[pl.ds(k * LSTM_BLK, LSTM_BLK), :]
        rows = []
        for j in range(LSTM_BLK):
            carry, hnew = step(carry, gblk[j:j + 1, :])
            rows.append(hnew)
        hs_ref[pl.ds(k * LSTM_BLK, LSTM_BLK), :] = jnp.concatenate(rows, axis=0)
        return carry

    zero = jnp.zeros((1, H), jnp.float32)
    jax.lax.fori_loop(0, N // LSTM_BLK, blk, (zero, zero), unroll=False)

    hs = hs_ref[...]
    scores = jnp.dot(hs, wa_ref[...], preferred_element_type=jnp.float32) + ba_ref[0, 0]
    m = jnp.max(scores)
    wexp = jnp.exp(scores - m)
    denom = jnp.sum(wexp)
    pooled = jnp.dot(wexp.T, hs, preferred_element_type=jnp.float32) / denom
    logits = jnp.dot(pooled, wf_ref[...], preferred_element_type=jnp.float32) + bf_ref[...]
    lmax = jnp.max(logits, axis=1, keepdims=True)
    lexp = jnp.exp(logits - lmax)
    out_ref[...] = lexp / jnp.sum(lexp, axis=1, keepdims=True)


def _lstm_head(acc, den, b2, WihT, WhhT, bsum, Wa, ba, Wf, bf):
    return pl.pallas_call(
        _lstm_head_body,
        out_shape=jax.ShapeDtypeStruct((1, C), jnp.float32),
        scratch_shapes=[
            pltpu.VMEM((N, 4 * H), jnp.float32),
            pltpu.VMEM((N, H), jnp.float32),
        ],
    )(acc, den, b2[None, :], WihT, WhhT, bsum, Wa, ba, Wf, bf)


def kernel(x, edge_index, W1, a_src1, a_dst1, b1, W2, a_src2, a_dst2, b2,
           Wih, Whh, bih, bhh, Wa, ba, Wf, bf):
    src = edge_index[0]
    dst = edge_index[1]
    z64 = jnp.zeros((NPD, H), jnp.float32)
    z1 = jnp.zeros((NPD,), jnp.float32)
    xp = jnp.pad(x, ((0, NPD - N), (0, 0)))

    h1, asrc1, adst1, m1 = _prep1(xp, W1, a_src1, a_dst1)
    acc1, den1 = _sc_gat_edges(src, dst, asrc1[:, 0], adst1[:, 0], h1,
                               m1[0], z64, z1)

    h2, asrc2, adst2, m2 = _prep2(acc1, den1, b1, W2, a_src2, a_dst2)
    acc2, den2 = _sc_gat_edges(src, dst, asrc2[:, 0], adst2[:, 0], h2,
                               m2[0], z64, z1)

    bsum = (bih + bhh)[None, :]
    out = _lstm_head(acc2, den2, b2, Wih.T, Whh.T, bsum, Wa, ba[:, None],
                     Wf, bf[None, :])
    return out
